# baseline (speedup 1.0000x reference)
import functools

import jax
import jax.numpy as jnp
from jax import lax
from jax.experimental import pallas as pl
from jax.experimental.pallas import tpu as pltpu
from jax.experimental.pallas import tpu_sc as plsc

_INFO = plsc.get_sparse_core_info()
_NC, _NS = _INFO.num_cores, _INFO.num_subcores
_NW = _NC * _NS

_CHUNK = 128
_NBUF = 4


def kernel(new_k, new_v):
    shape = new_k.shape
    k2 = new_k.reshape(-1, shape[-1])
    v2 = new_v.reshape(-1, shape[-1])
    rows, cols = k2.shape
    rows_per = rows // _NW
    nch = rows_per // _CHUNK

    mesh = plsc.VectorSubcoreMesh(core_axis_name="c", subcore_axis_name="s")

    @functools.partial(
        pl.kernel,
        mesh=mesh,
        out_type=[
            jax.ShapeDtypeStruct(k2.shape, k2.dtype),
            jax.ShapeDtypeStruct(v2.shape, v2.dtype),
        ],
        scratch_types=[
            pltpu.VMEM((_NBUF, _CHUNK, 128), jnp.float32),
            pltpu.SemaphoreType.DMA((_NBUF,)),
            pltpu.SemaphoreType.DMA((_NBUF,)),
        ],
    )
    def _sc_copy(k_hbm, v_hbm, k_out, v_out, bufs, sin, sout):
        wid = lax.axis_index("s") * _NC + lax.axis_index("c")
        base = wid * rows_per

        seq = [(k_hbm, k_out, i) for i in range(nch)]
        seq += [(v_hbm, v_out, i) for i in range(nch)]
        n = len(seq)

        def in_copy(t):
            src, _, i = seq[t]
            b = t % _NBUF
            sl = pl.ds(base + i * _CHUNK, _CHUNK)
            return pltpu.make_async_copy(src.at[sl], bufs.at[b], sin.at[b])

        def out_copy(t):
            _, dst, i = seq[t]
            b = t % _NBUF
            sl = pl.ds(base + i * _CHUNK, _CHUNK)
            return pltpu.make_async_copy(bufs.at[b], dst.at[sl], sout.at[b])

        in_copy(0).start()
        in_copy(1).start()
        for t in range(n):
            in_copy(t).wait()
            out_copy(t).start()
            if t + 2 < n:
                if t >= 2:
                    out_copy(t - 2).wait()
                in_copy(t + 2).start()
        out_copy(n - 2).wait()
        out_copy(n - 1).wait()

    out = _sc_copy(k2, v2)
    return (out[0].reshape(shape), out[1].reshape(shape))


# speedup vs baseline: 1.2457x; 1.2457x over previous
import jax
import jax.numpy as jnp
from jax.experimental import pallas as pl
from jax.experimental.pallas import tpu as pltpu


_BLOCK_ROWS = 8192


def _copy_body(k_in, v_in, k_out, v_out):
    k_out[...] = k_in[...]
    v_out[...] = v_in[...]


def kernel(new_k, new_v):
    shape = new_k.shape
    k2 = new_k.reshape(-1, shape[-1])
    v2 = new_v.reshape(-1, shape[-1])
    rows, cols = k2.shape
    grid = (rows // _BLOCK_ROWS,)
    spec = pl.BlockSpec((_BLOCK_ROWS, cols), lambda i: (i, 0))
    out = pl.pallas_call(
        _copy_body,
        grid=grid,
        in_specs=[spec, spec],
        out_specs=[spec, spec],
        out_shape=[
            jax.ShapeDtypeStruct(k2.shape, k2.dtype),
            jax.ShapeDtypeStruct(v2.shape, v2.dtype),
        ],
        compiler_params=pltpu.CompilerParams(
            dimension_semantics=("parallel",),
            skip_device_barrier=True,
            disable_bounds_checks=True,
        ),
    )(k2, v2)
    return (out[0].reshape(shape), out[1].reshape(shape))
